# Initial kernel scaffold; baseline (speedup 1.0000x reference)
import jax
import jax.numpy as jnp
from jax.experimental import pallas as pl


def kernel(sensor_id, table):
    raise NotImplementedError("write your pallas kernel here")


# speedup vs baseline: 4.8098x; 4.8098x over previous
import functools

import jax
import jax.numpy as jnp
from jax import lax
from jax.experimental import pallas as pl
from jax.experimental.pallas import tpu as pltpu
from jax.experimental.pallas import tpu_sc as plsc

D_EMBED = 32
CHUNK = 1024


@functools.lru_cache(maxsize=None)
def _make_gather(B: int):
    info = plsc.get_sparse_core_info()
    nc, ns = info.num_cores, info.num_subcores
    nw = nc * ns
    b_per_w = B // nw
    assert b_per_w * nw == B
    n_chunks = b_per_w // CHUNK
    assert n_chunks * CHUNK == b_per_w

    mesh = plsc.VectorSubcoreMesh(core_axis_name="c", subcore_axis_name="s")

    @functools.partial(
        pl.kernel,
        mesh=mesh,
        out_type=jax.ShapeDtypeStruct((B, D_EMBED), jnp.float32),
        scratch_types=[
            pltpu.VMEM((CHUNK,), jnp.int32),
            pltpu.VMEM((CHUNK, D_EMBED), jnp.float32),
            pltpu.SemaphoreType.DMA,
        ],
        compiler_params=pltpu.CompilerParams(use_tc_tiling_on_sc=False),
    )
    def gather_kernel(idx_hbm, table_hbm, out_hbm, idx_v, rows_v, sem):
        wid = lax.axis_index("s") * nc + lax.axis_index("c")
        base = wid * b_per_w

        def body(i, carry):
            off = base + i * CHUNK
            pltpu.sync_copy(idx_hbm.at[pl.ds(off, CHUNK)], idx_v)
            pltpu.async_copy(table_hbm.at[idx_v], rows_v, sem).wait()
            pltpu.sync_copy(rows_v, out_hbm.at[pl.ds(off, CHUNK)])
            return carry

        lax.fori_loop(0, n_chunks, body, 0)

    return gather_kernel


def kernel(sensor_id, table):
    s, t = sensor_id.shape
    flat = sensor_id.reshape(s * t)
    out = _make_gather(s * t)(flat, table)
    return out.reshape(s, t, D_EMBED)


# speedup vs baseline: 5.0402x; 1.0479x over previous
import functools

import jax
import jax.numpy as jnp
from jax import lax
from jax.experimental import pallas as pl
from jax.experimental.pallas import tpu as pltpu
from jax.experimental.pallas import tpu_sc as plsc

D_EMBED = 32
CHUNK = 1280
NBUF = 2


@functools.lru_cache(maxsize=None)
def _make_gather(B: int):
    info = plsc.get_sparse_core_info()
    nc, ns = info.num_cores, info.num_subcores
    nw = nc * ns
    b_per_w = B // nw
    assert b_per_w * nw == B
    n_chunks = b_per_w // CHUNK
    assert n_chunks * CHUNK == b_per_w and n_chunks % NBUF == 0
    n_groups = n_chunks // NBUF

    mesh = plsc.VectorSubcoreMesh(core_axis_name="c", subcore_axis_name="s")

    @functools.partial(
        pl.kernel,
        mesh=mesh,
        out_type=jax.ShapeDtypeStruct((B, D_EMBED), jnp.float32),
        scratch_types=(
            [pltpu.VMEM((CHUNK,), jnp.int32) for _ in range(NBUF)]
            + [pltpu.VMEM((CHUNK, D_EMBED), jnp.float32) for _ in range(NBUF)]
            + [pltpu.SemaphoreType.DMA for _ in range(3 * NBUF)]
        ),
        compiler_params=pltpu.CompilerParams(use_tc_tiling_on_sc=False),
    )
    def gather_kernel(idx_hbm, table_hbm, out_hbm, *scratch):
        idx_v = scratch[:NBUF]
        rows_v = scratch[NBUF:2 * NBUF]
        sem_idx = scratch[2 * NBUF:3 * NBUF]
        sem_g = scratch[3 * NBUF:4 * NBUF]
        sem_st = scratch[4 * NBUF:5 * NBUF]

        wid = lax.axis_index("s") * nc + lax.axis_index("c")
        base = wid * b_per_w

        def idx_slice(i):
            return idx_hbm.at[pl.ds(base + i * CHUNK, CHUNK)]

        def out_slice(i):
            return out_hbm.at[pl.ds(base + i * CHUNK, CHUNK)]

        for b in range(NBUF):
            pltpu.async_copy(idx_slice(b), idx_v[b], sem_idx[b])

        def group(g, carry):
            for b in range(NBUF):
                i = g * NBUF + b
                pltpu.make_async_copy(idx_slice(0), idx_v[b], sem_idx[b]).wait()
                @pl.when(g > 0)
                def _():
                    pltpu.make_async_copy(rows_v[b], out_slice(0), sem_st[b]).wait()
                pltpu.async_copy(table_hbm.at[idx_v[b]], rows_v[b], sem_g[b]).wait()
                pltpu.async_copy(rows_v[b], out_slice(i), sem_st[b])
                @pl.when(i + NBUF < n_chunks)
                def _():
                    pltpu.async_copy(idx_slice(i + NBUF), idx_v[b], sem_idx[b])
            return carry

        lax.fori_loop(0, n_groups, group, 0)

        for b in range(NBUF):
            pltpu.make_async_copy(rows_v[b], out_slice(0), sem_st[b]).wait()

    return gather_kernel


def kernel(sensor_id, table):
    s, t = sensor_id.shape
    flat = sensor_id.reshape(s * t)
    out = _make_gather(s * t)(flat, table)
    return out.reshape(s, t, D_EMBED)
